# baseline (speedup 1.0000x reference)
import functools

import jax
import jax.numpy as jnp
from jax import lax
from jax.experimental import pallas as pl
from jax.experimental.pallas import tpu as pltpu
from jax.experimental.pallas import tpu_sc as plsc

DIM = 4096
NUM_EXPERTS = 64
TOP_K = 8
TOKENS_PER_BLOCK = 1024

_NEG_INF = float("-inf")


def _matmul_block(x_ref, w_ref, logits_ref, logits_t_ref):
    logits_t = lax.dot_general(
        w_ref[:], x_ref[:], (((1,), (1,)), ((), ())),
        preferred_element_type=jnp.float32,
    )
    logits_t_ref[:] = logits_t
    logits_ref[:] = logits_t.T


def _tc_logits(xt, W):
    n_tokens = xt.shape[0]
    n_blocks = n_tokens // TOKENS_PER_BLOCK
    return pl.pallas_call(
        _matmul_block,
        grid=(n_blocks,),
        in_specs=[
            pl.BlockSpec((TOKENS_PER_BLOCK, DIM), lambda i: (i, 0)),
            pl.BlockSpec((NUM_EXPERTS, DIM), lambda i: (0, 0)),
        ],
        out_specs=[
            pl.BlockSpec((TOKENS_PER_BLOCK, NUM_EXPERTS), lambda i: (i, 0)),
            pl.BlockSpec((NUM_EXPERTS, TOKENS_PER_BLOCK), lambda i: (0, i)),
        ],
        out_shape=[
            jax.ShapeDtypeStruct((n_tokens, NUM_EXPERTS), jnp.float32),
            jax.ShapeDtypeStruct((NUM_EXPERTS, n_tokens), jnp.float32),
        ],
    )(xt, W)


def _sc_topk(logits_t):
    n_tokens = logits_t.shape[1]
    info = plsc.get_sparse_core_info()
    nw = info.num_cores * info.num_subcores
    chunk = n_tokens // nw
    n_groups = chunk // 16

    @functools.partial(
        pl.kernel,
        mesh=plsc.VectorSubcoreMesh(core_axis_name="c", subcore_axis_name="s"),
        out_type=[
            jax.ShapeDtypeStruct((TOP_K, n_tokens), jnp.float32),
            jax.ShapeDtypeStruct((TOP_K, n_tokens), jnp.int32),
        ],
        scratch_types=[
            pltpu.VMEM((NUM_EXPERTS, chunk), jnp.float32),
            pltpu.VMEM((TOP_K, chunk), jnp.float32),
            pltpu.VMEM((TOP_K, chunk), jnp.int32),
        ],
    )
    def topk_kernel(lt_hbm, w_hbm, i_hbm, lt_v, w_v, i_v):
        wid = lax.axis_index("s") * info.num_cores + lax.axis_index("c")
        base = wid * chunk
        pltpu.sync_copy(lt_hbm.at[:, pl.ds(base, chunk)], lt_v)

        def group_body(g, carry):
            off = g * 16
            sl = pl.ds(off, 16)
            rows = [lt_v[e, sl] for e in range(NUM_EXPERTS)]
            vals = []
            idxs = []
            for _ in range(TOP_K):
                ent = [
                    (rows[e], jnp.full((16,), e, jnp.int32))
                    for e in range(NUM_EXPERTS)
                ]
                while len(ent) > 1:
                    nxt = []
                    for j in range(0, len(ent), 2):
                        (va, ia), (vb, ib) = ent[j], ent[j + 1]
                        p = va >= vb
                        nxt.append((jnp.where(p, va, vb), jnp.where(p, ia, ib)))
                    ent = nxt
                m, mi = ent[0]
                vals.append(m)
                idxs.append(mi)
                rows = [
                    jnp.where(mi == e, _NEG_INF, rows[e])
                    for e in range(NUM_EXPERTS)
                ]
            exps = [jnp.exp(v - vals[0]) for v in vals]
            tot = exps[0]
            for e2 in exps[1:]:
                tot = tot + e2
            for k in range(TOP_K):
                w_v[k, sl] = exps[k] / tot
                i_v[k, sl] = idxs[k]
            return carry

        lax.fori_loop(0, n_groups, group_body, 0)
        pltpu.sync_copy(w_v, w_hbm.at[:, pl.ds(base, chunk)])
        pltpu.sync_copy(i_v, i_hbm.at[:, pl.ds(base, chunk)])

    return topk_kernel(logits_t)


N_CHUNKS = 4


@jax.jit
def kernel(x, W):
    b, s, d = x.shape
    n_tokens = b * s
    xt = x.reshape(n_tokens, d)

    chunk = n_tokens // N_CHUNKS
    logits_c, wt_c, it_c = [], [], []
    for c in range(N_CHUNKS):
        lg, lt = _tc_logits(lax.dynamic_slice_in_dim(xt, c * chunk, chunk), W)
        w_t, i_t = _sc_topk(lt)
        logits_c.append(lg)
        wt_c.append(w_t)
        it_c.append(i_t)

    logits = jnp.concatenate(logits_c, axis=0)
    w_t = jnp.concatenate(wt_c, axis=1)
    i_t = jnp.concatenate(it_c, axis=1)
    return (
        w_t.T.reshape(b, s, TOP_K),
        i_t.T.reshape(b, s, TOP_K),
        logits.reshape(b, s, NUM_EXPERTS),
    )


# speedup vs baseline: 2.7877x; 2.7877x over previous
import functools

import jax
import jax.numpy as jnp
from jax import lax
from jax.experimental import pallas as pl

DIM = 4096
NUM_EXPERTS = 64
TOP_K = 8
TOKENS_PER_BLOCK = 1024
HALF = TOKENS_PER_BLOCK // 2

_NEG_INF = float("-inf")


def _router_block(x0_ref, x1_ref, w_ref, logits_ref, weights_ref, indices_ref):
    t0 = lax.dot_general(
        w_ref[:], x0_ref[:], (((1,), (1,)), ((), ())),
        preferred_element_type=jnp.float32,
    )
    t1 = lax.dot_general(
        w_ref[:], x1_ref[:], (((1,), (1,)), ((), ())),
        preferred_element_type=jnp.float32,
    )
    logits_t = jnp.concatenate([t0, t1], axis=1)
    logits_ref[:] = logits_t.T

    t = logits_t.shape[1]
    iota = lax.broadcasted_iota(jnp.int32, (NUM_EXPERTS, t), 0)
    cur = logits_t
    vals = []
    idxs = []
    for _ in range(TOP_K):
        m = jnp.max(cur, axis=0, keepdims=True)
        idx = jnp.min(jnp.where(cur == m, iota, NUM_EXPERTS), axis=0, keepdims=True)
        vals.append(m)
        idxs.append(idx)
        cur = jnp.where(iota == idx, _NEG_INF, cur)
    vals = jnp.concatenate(vals, axis=0)
    idxs = jnp.concatenate(idxs, axis=0)

    e = jnp.exp(vals - vals[:1])
    weights_ref[:] = (e / jnp.sum(e, axis=0, keepdims=True)).T
    indices_ref[:] = idxs.T


@jax.jit
def kernel(x, W):
    b, s, d = x.shape
    n_tokens = b * s
    xt = x.reshape(n_tokens, d)

    n_blocks = n_tokens // TOKENS_PER_BLOCK
    logits, weights, indices = pl.pallas_call(
        _router_block,
        grid=(n_blocks,),
        in_specs=[
            pl.BlockSpec((HALF, d), lambda i: (2 * i, 0)),
            pl.BlockSpec((HALF, d), lambda i: (2 * i + 1, 0)),
            pl.BlockSpec((NUM_EXPERTS, d), lambda i: (0, 0)),
        ],
        out_specs=[
            pl.BlockSpec((TOKENS_PER_BLOCK, NUM_EXPERTS), lambda i: (i, 0)),
            pl.BlockSpec((TOKENS_PER_BLOCK, TOP_K), lambda i: (i, 0)),
            pl.BlockSpec((TOKENS_PER_BLOCK, TOP_K), lambda i: (i, 0)),
        ],
        out_shape=[
            jax.ShapeDtypeStruct((n_tokens, NUM_EXPERTS), jnp.float32),
            jax.ShapeDtypeStruct((n_tokens, TOP_K), jnp.float32),
            jax.ShapeDtypeStruct((n_tokens, TOP_K), jnp.int32),
        ],
    )(xt, xt, W)

    return (
        weights.reshape(b, s, TOP_K),
        indices.reshape(b, s, TOP_K),
        logits.reshape(b, s, NUM_EXPERTS),
    )
